# baseline (speedup 1.0000x reference)
import functools

import jax
import jax.numpy as jnp
from jax import lax
from jax.experimental import pallas as pl
from jax.experimental.pallas import tpu as pltpu
from jax.experimental.pallas import tpu_sc as plsc

_NC = 2
_NS = 16
_NW = _NC * _NS

_CHUNK = 1024
_BLK = 8192


def _make_gather(tok, hid):
    assert tok % (_NW * _CHUNK) == 0
    tok_per_w = tok // _NW
    n_chunk = tok_per_w // _CHUNK
    mesh = plsc.VectorSubcoreMesh(core_axis_name="c", subcore_axis_name="s")

    @functools.partial(
        pl.kernel,
        out_type=jax.ShapeDtypeStruct((tok, hid), jnp.float32),
        mesh=mesh,
        scratch_types=[
            pltpu.VMEM((_CHUNK,), jnp.int32),
            pltpu.VMEM((_CHUNK, hid), jnp.float32),
            pltpu.SemaphoreType.DMA,
        ],
        compiler_params=pltpu.CompilerParams(use_tc_tiling_on_sc=False),
    )
    def gather(idx_hbm, table_hbm, out_hbm, idx_v, rows_v, sem):
        wid = lax.axis_index("s") * _NC + lax.axis_index("c")
        base = wid * tok_per_w

        def body(i, carry):
            off = base + i * _CHUNK
            pltpu.sync_copy(idx_hbm.at[pl.ds(off, _CHUNK)], idx_v)
            pltpu.async_copy(table_hbm.at[idx_v], rows_v, sem).wait()
            pltpu.sync_copy(rows_v, out_hbm.at[pl.ds(off, _CHUNK)])
            return carry

        lax.fori_loop(0, n_chunk, body, 0)

    return gather


def _make_transpose(n_emb, hid):
    assert hid == 32
    n_blk = n_emb // 128
    tail = n_emb - n_blk * 128
    mesh = plsc.VectorSubcoreMesh(core_axis_name="c", subcore_axis_name="s")

    @functools.partial(
        pl.kernel,
        out_type=jax.ShapeDtypeStruct((n_emb // 4, 128), jnp.float32),
        mesh=mesh,
        scratch_types=[
            pltpu.VMEM((32, 128), jnp.float32),
            pltpu.VMEM((32, 128), jnp.float32),
            pltpu.VMEM((16, 128), jnp.float32),
        ],
        compiler_params=pltpu.CompilerParams(needs_layout_passes=False),
    )
    def transpose(emt_hbm, tail_hbm, out_hbm, in_v, out_v, tail_v):
        wid = lax.axis_index("s") * _NC + lax.axis_index("c")
        lane = lax.iota(jnp.int32, 16)
        lane16 = lane + 16

        def do_block(b, carry):
            pltpu.sync_copy(emt_hbm.at[:, pl.ds(b * 128, 128)], in_v)
            def row_q(q, c):
                for s in range(4):
                    r = q * 4 + s
                    rv = jnp.full((16,), r, jnp.int32)
                    v0 = plsc.load_gather(in_v, [lane, rv])
                    v1 = plsc.load_gather(in_v, [lane16, rv])
                    out_v[q, pl.ds(32 * s, 16)] = v0
                    out_v[q, pl.ds(32 * s + 16, 16)] = v1
                return c

            lax.fori_loop(0, 32, row_q, 0)
            pltpu.sync_copy(out_v, out_hbm.at[pl.ds(b * 32, 32)])
            return carry

        def strided(i, carry):
            return do_block(i * _NW + wid, carry)

        lax.fori_loop(0, n_blk // _NW, strided, 0)
        rem = n_blk % _NW
        if rem:
            @pl.when(wid < rem)
            def _():
                do_block((n_blk // _NW) * _NW + wid, 0)

        if tail:
            @pl.when(wid == 0)
            def _():
                pltpu.sync_copy(tail_hbm, tail_v.at[pl.ds(0, tail // 4)])
                pltpu.sync_copy(
                    tail_v.at[pl.ds(0, tail // 4)],
                    out_hbm.at[pl.ds(n_blk * 32, tail // 4)],
                )

    return transpose


def _matmul_body(emb_ref, w4_ref, out_ref):
    res = jnp.dot(emb_ref[...], w4_ref[...], preferred_element_type=jnp.float32)
    for i in range(4):
        out_ref[pl.ds(i * _BLK, _BLK), :] = res[:, i * 128:(i + 1) * 128]


def _project(emb_wide, w4, tok, emb_dim):
    rows = emb_wide.shape[0]
    grid = rows // _BLK
    return pl.pallas_call(
        _matmul_body,
        grid=(grid,),
        in_specs=[
            pl.BlockSpec((_BLK, 128), lambda i: (i, 0)),
            pl.BlockSpec((128, 4 * 128), lambda i: (0, 0)),
        ],
        out_specs=pl.BlockSpec((4 * _BLK, emb_dim), lambda i: (i, 0)),
        out_shape=jax.ShapeDtypeStruct((tok, emb_dim), jnp.float32),
    )(emb_wide, w4)


def kernel(x, em_weight, fc_weight):
    b, seq = x.shape
    n_emb, hid = em_weight.shape
    emb_dim = fc_weight.shape[0]
    tok = b * seq
    grp = 4 * _BLK

    idx = x.reshape(tok // grp, 4, _BLK).transpose(0, 2, 1).reshape(tok)

    n_tail = n_emb % 128
    tail = lax.slice(em_weight, (n_emb - n_tail, 0), (n_emb, hid))
    tail_wide = tail.reshape(n_tail // 4, 4 * hid)
    table_wide = _make_transpose(n_emb, hid)(em_weight.T, tail_wide)
    table = table_wide.reshape(n_emb, hid)

    emb = _make_gather(tok, hid)(idx, table)
    emb_wide = emb.reshape(tok // 4, 4 * hid)

    fct = fc_weight.T
    eye4 = jnp.eye(4, dtype=fct.dtype)
    w4 = jnp.einsum("gh,ke->gkhe", eye4, fct).reshape(4 * hid, 4 * emb_dim)

    out = _project(emb_wide, w4, tok, emb_dim)
    return out.reshape(b, seq, emb_dim)


# speedup vs baseline: 1.5539x; 1.5539x over previous
import functools

import jax
import jax.numpy as jnp
from jax import lax
from jax.experimental import pallas as pl
from jax.experimental.pallas import tpu as pltpu
from jax.experimental.pallas import tpu_sc as plsc

_NC = 2
_NS = 16
_NW = _NC * _NS

_CHUNK = 2560
_BLK = 8192


def _make_gather(tok, hid):
    assert tok % (_NW * _CHUNK) == 0
    tok_per_w = tok // _NW
    n_chunk = tok_per_w // _CHUNK
    mesh = plsc.VectorSubcoreMesh(core_axis_name="c", subcore_axis_name="s")

    @functools.partial(
        pl.kernel,
        out_type=jax.ShapeDtypeStruct((tok, hid), jnp.float32),
        mesh=mesh,
        scratch_types=[
            pltpu.VMEM((_CHUNK,), jnp.int32),
            pltpu.VMEM((_CHUNK, hid), jnp.float32),
            pltpu.SemaphoreType.DMA,
        ],
        compiler_params=pltpu.CompilerParams(use_tc_tiling_on_sc=False),
    )
    def gather(idx_hbm, table_hbm, out_hbm, idx_v, rows_v, sem):
        wid = lax.axis_index("s") * _NC + lax.axis_index("c")
        base = wid * tok_per_w

        def body(i, carry):
            off = base + i * _CHUNK
            pltpu.sync_copy(idx_hbm.at[pl.ds(off, _CHUNK)], idx_v)
            pltpu.async_copy(table_hbm.at[idx_v], rows_v, sem).wait()
            pltpu.sync_copy(rows_v, out_hbm.at[pl.ds(off, _CHUNK)])
            return carry

        lax.fori_loop(0, n_chunk, body, 0)

    return gather


def _matmul_body(emb_ref, w4_ref, out_ref):
    res = jnp.dot(emb_ref[...], w4_ref[...], preferred_element_type=jnp.float32)
    for i in range(4):
        out_ref[pl.ds(i * _BLK, _BLK), :] = res[:, i * 128:(i + 1) * 128]


def _project(emb_wide, w4, tok, emb_dim):
    rows = emb_wide.shape[0]
    grid = rows // _BLK
    return pl.pallas_call(
        _matmul_body,
        grid=(grid,),
        in_specs=[
            pl.BlockSpec((_BLK, 128), lambda i: (i, 0)),
            pl.BlockSpec((128, 4 * 128), lambda i: (0, 0)),
        ],
        out_specs=pl.BlockSpec((4 * _BLK, emb_dim), lambda i: (i, 0)),
        out_shape=jax.ShapeDtypeStruct((tok, emb_dim), jnp.float32),
    )(emb_wide, w4)


def kernel(x, em_weight, fc_weight):
    b, seq = x.shape
    n_emb, hid = em_weight.shape
    emb_dim = fc_weight.shape[0]
    tok = b * seq
    grp = 4 * _BLK

    idx = x.reshape(tok // grp, 4, _BLK).transpose(0, 2, 1).reshape(tok)

    emb = _make_gather(tok, hid)(idx, em_weight)
    emb_wide = emb.reshape(tok // 4, 4 * hid)

    fct = fc_weight.T
    eye4 = jnp.eye(4, dtype=fct.dtype)
    w4 = jnp.einsum("gh,ke->gkhe", eye4, fct).reshape(4 * hid, 4 * emb_dim)

    out = _project(emb_wide, w4, tok, emb_dim)
    return out.reshape(b, seq, emb_dim)


# speedup vs baseline: 1.5612x; 1.0047x over previous
import functools

import jax
import jax.numpy as jnp
from jax import lax
from jax.experimental import pallas as pl
from jax.experimental.pallas import tpu as pltpu
from jax.experimental.pallas import tpu_sc as plsc

_NC = 2
_NS = 16
_NW = _NC * _NS

_CHUNK = 1280
_BLK = 8192


def _make_gather(tok, hid):
    assert tok % (_NW * _CHUNK) == 0
    tok_per_w = tok // _NW
    n_chunk = tok_per_w // _CHUNK
    mesh = plsc.VectorSubcoreMesh(core_axis_name="c", subcore_axis_name="s")

    @functools.partial(
        pl.kernel,
        out_type=jax.ShapeDtypeStruct((tok, hid), jnp.float32),
        mesh=mesh,
        scratch_types=[
            pltpu.VMEM((tok_per_w,), jnp.int32),
            pltpu.VMEM((_CHUNK, hid), jnp.float32),
            pltpu.VMEM((_CHUNK, hid), jnp.float32),
            pltpu.SemaphoreType.DMA,
            pltpu.SemaphoreType.DMA,
            pltpu.SemaphoreType.DMA,
            pltpu.SemaphoreType.DMA,
        ],
        compiler_params=pltpu.CompilerParams(use_tc_tiling_on_sc=False),
    )
    def gather(idx_hbm, table_hbm, out_hbm, idx_v, rows_v0, rows_v1,
               sg0, sg1, sw0, sw1):
        wid = lax.axis_index("s") * _NC + lax.axis_index("c")
        base = wid * tok_per_w
        pltpu.sync_copy(idx_hbm.at[pl.ds(base, tok_per_w)], idx_v)

        rows = (rows_v0, rows_v1)
        sg = (sg0, sg1)
        sw = (sw0, sw1)
        writes = [None, None]
        for i in range(n_chunk):
            b = i % 2
            if writes[b] is not None:
                writes[b].wait()
            g = pltpu.async_copy(
                table_hbm.at[idx_v.at[pl.ds(i * _CHUNK, _CHUNK)]],
                rows[b], sg[b],
            )
            g.wait()
            writes[b] = pltpu.async_copy(
                rows[b], out_hbm.at[pl.ds(base + i * _CHUNK, _CHUNK)], sw[b]
            )
        for w in writes:
            if w is not None:
                w.wait()

    return gather


def _matmul_body(emb_ref, w4_ref, out_ref):
    res = jnp.dot(emb_ref[...], w4_ref[...], preferred_element_type=jnp.float32)
    for i in range(4):
        out_ref[pl.ds(i * _BLK, _BLK), :] = res[:, i * 128:(i + 1) * 128]


def _project(emb_wide, w4, tok, emb_dim):
    rows = emb_wide.shape[0]
    grid = rows // _BLK
    return pl.pallas_call(
        _matmul_body,
        grid=(grid,),
        in_specs=[
            pl.BlockSpec((_BLK, 128), lambda i: (i, 0)),
            pl.BlockSpec((128, 4 * 128), lambda i: (0, 0)),
        ],
        out_specs=pl.BlockSpec((4 * _BLK, emb_dim), lambda i: (i, 0)),
        out_shape=jax.ShapeDtypeStruct((tok, emb_dim), jnp.float32),
    )(emb_wide, w4)


def kernel(x, em_weight, fc_weight):
    b, seq = x.shape
    n_emb, hid = em_weight.shape
    emb_dim = fc_weight.shape[0]
    tok = b * seq
    grp = 4 * _BLK

    idx = x.reshape(tok // grp, 4, _BLK).transpose(0, 2, 1).reshape(tok)

    emb = _make_gather(tok, hid)(idx, em_weight)
    emb_wide = emb.reshape(tok // 4, 4 * hid)

    fct = fc_weight.T
    eye4 = jnp.eye(4, dtype=fct.dtype)
    w4 = jnp.einsum("gh,ke->gkhe", eye4, fct).reshape(4 * hid, 4 * emb_dim)

    out = _project(emb_wide, w4, tok, emb_dim)
    return out.reshape(b, seq, emb_dim)


# speedup vs baseline: 1.5615x; 1.0002x over previous
import functools

import jax
import jax.numpy as jnp
from jax import lax
from jax.experimental import pallas as pl
from jax.experimental.pallas import tpu as pltpu
from jax.experimental.pallas import tpu_sc as plsc

_NC = 2
_NS = 16
_NW = _NC * _NS

_CHUNK = 1280
_BLK = 10240


def _make_gather(tok, hid):
    assert tok % (_NW * _CHUNK) == 0
    tok_per_w = tok // _NW
    n_chunk = tok_per_w // _CHUNK
    mesh = plsc.VectorSubcoreMesh(core_axis_name="c", subcore_axis_name="s")

    @functools.partial(
        pl.kernel,
        out_type=jax.ShapeDtypeStruct((tok, hid), jnp.float32),
        mesh=mesh,
        scratch_types=[
            pltpu.VMEM((tok_per_w,), jnp.int32),
            pltpu.VMEM((_CHUNK, hid), jnp.float32),
            pltpu.VMEM((_CHUNK, hid), jnp.float32),
            pltpu.SemaphoreType.DMA,
            pltpu.SemaphoreType.DMA,
            pltpu.SemaphoreType.DMA,
            pltpu.SemaphoreType.DMA,
        ],
        compiler_params=pltpu.CompilerParams(use_tc_tiling_on_sc=False),
    )
    def gather(idx_hbm, table_hbm, out_hbm, idx_v, rows_v0, rows_v1,
               sg0, sg1, sw0, sw1):
        wid = lax.axis_index("s") * _NC + lax.axis_index("c")
        base = wid * tok_per_w
        pltpu.sync_copy(idx_hbm.at[pl.ds(base, tok_per_w)], idx_v)

        rows = (rows_v0, rows_v1)
        sg = (sg0, sg1)
        sw = (sw0, sw1)
        writes = [None, None]
        for i in range(n_chunk):
            b = i % 2
            if writes[b] is not None:
                writes[b].wait()
            g = pltpu.async_copy(
                table_hbm.at[idx_v.at[pl.ds(i * _CHUNK, _CHUNK)]],
                rows[b], sg[b],
            )
            g.wait()
            writes[b] = pltpu.async_copy(
                rows[b], out_hbm.at[pl.ds(base + i * _CHUNK, _CHUNK)], sw[b]
            )
        for w in writes:
            if w is not None:
                w.wait()

    return gather


def _matmul_body(emb_ref, w4_ref, out_ref):
    res = jnp.dot(emb_ref[...], w4_ref[...], preferred_element_type=jnp.float32)
    for i in range(4):
        out_ref[pl.ds(i * _BLK, _BLK), :] = res[:, i * 128:(i + 1) * 128]


def _project(emb_wide, w4, tok, emb_dim):
    rows = emb_wide.shape[0]
    grid = rows // _BLK
    return pl.pallas_call(
        _matmul_body,
        grid=(grid,),
        in_specs=[
            pl.BlockSpec((_BLK, 128), lambda i: (i, 0)),
            pl.BlockSpec((128, 4 * 128), lambda i: (0, 0)),
        ],
        out_specs=pl.BlockSpec((4 * _BLK, emb_dim), lambda i: (i, 0)),
        out_shape=jax.ShapeDtypeStruct((tok, emb_dim), jnp.float32),
    )(emb_wide, w4)


def kernel(x, em_weight, fc_weight):
    b, seq = x.shape
    n_emb, hid = em_weight.shape
    emb_dim = fc_weight.shape[0]
    tok = b * seq
    grp = 4 * _BLK

    idx = x.reshape(tok // grp, 4, _BLK).transpose(0, 2, 1).reshape(tok)

    emb = _make_gather(tok, hid)(idx, em_weight)
    emb_wide = emb.reshape(tok // 4, 4 * hid)

    fct = fc_weight.T
    eye4 = jnp.eye(4, dtype=fct.dtype)
    w4 = jnp.einsum("gh,ke->gkhe", eye4, fct).reshape(4 * hid, 4 * emb_dim)

    out = _project(emb_wide, w4, tok, emb_dim)
    return out.reshape(b, seq, emb_dim)
